# baseline (speedup 1.0000x reference)
import functools

import jax
import jax.numpy as jnp
from jax import lax
from jax.experimental import pallas as pl
from jax.experimental.pallas import tpu as pltpu

N = 10000
E = 160000
F = 128
DIM = 32
NG = 128
EB = 640
N_TILES = E // EB

_f32 = jnp.float32


def _init_body(x_ref, w_ref, b_ref, o_ref):
    o_ref[...] = jnp.maximum(
        jnp.dot(x_ref[...], w_ref[...], preferred_element_type=_f32) + b_ref[...], 0.0)


def _edge_body(ea_ref, xj_ref, we1t_ref, be1_ref, we2t_ref, be2_ref, msg_ref):
    rh = jnp.maximum(
        jnp.dot(ea_ref[...], we1t_ref[...], preferred_element_type=_f32) + be1_ref[...], 0.0)
    p = jnp.dot(rh, we2t_ref[...], preferred_element_type=_f32) + be2_ref[...]
    xj = xj_ref[...]
    p3 = p.reshape(EB, DIM, DIM)
    msg_ref[...] = jnp.sum(p3 * xj[:, :, None], axis=1)


def _node_body(agg_ref, cnt_ref, out_ref, h_ref, wroot_ref, bconv_ref,
               wiht_ref, bih_ref, whht_ref, bhh_ref, hnew_ref):
    cnt = jnp.maximum(cnt_ref[...], 1.0)
    out = out_ref[...]
    h = h_ref[...]
    m = jnp.maximum(
        agg_ref[...] / cnt
        + jnp.dot(out, wroot_ref[...], preferred_element_type=_f32)
        + bconv_ref[...], 0.0)
    gi = jnp.dot(m, wiht_ref[...], preferred_element_type=_f32) + bih_ref[...]
    gh = jnp.dot(h, whht_ref[...], preferred_element_type=_f32) + bhh_ref[...]
    r = jax.nn.sigmoid(gi[:, 0:DIM] + gh[:, 0:DIM])
    z = jax.nn.sigmoid(gi[:, DIM:2 * DIM] + gh[:, DIM:2 * DIM])
    ng = jnp.tanh(gi[:, 2 * DIM:3 * DIM] + r * gh[:, 2 * DIM:3 * DIM])
    hnew_ref[...] = (1.0 - z) * ng + z * h


def _set2set_body(out_ref, batch_ref, wihst_ref, bihs_ref, whhst_ref, bhhs_ref,
                  w1t_ref, b1_ref, w2t_ref, b2_ref, ge_ref, pred_ref):
    out = out_ref[...]
    mask = batch_ref[...] == lax.broadcasted_iota(jnp.int32, (1, NG), 1)
    maskf = mask.astype(_f32)
    q_star = jnp.zeros((NG, 2 * DIM), dtype=_f32)
    hs = jnp.zeros((NG, DIM), dtype=_f32)
    cs = jnp.zeros((NG, DIM), dtype=_f32)
    for _ in range(3):
        gates = (jnp.dot(q_star, wihst_ref[...], preferred_element_type=_f32) + bihs_ref[...]
                 + jnp.dot(hs, whhst_ref[...], preferred_element_type=_f32) + bhhs_ref[...])
        gi = gates[:, 0:DIM]
        gf = gates[:, DIM:2 * DIM]
        gg = gates[:, 2 * DIM:3 * DIM]
        go = gates[:, 3 * DIM:4 * DIM]
        cs = jax.nn.sigmoid(gf) * cs + jax.nn.sigmoid(gi) * jnp.tanh(gg)
        hs = jax.nn.sigmoid(go) * jnp.tanh(cs)
        q = hs
        qb = jnp.dot(maskf, q, preferred_element_type=_f32)
        e = jnp.sum(out * qb, axis=1, keepdims=True)
        em = jnp.max(jnp.where(mask, e, -1e30), axis=0, keepdims=True)
        em = jnp.where(em > -1e29, em, 0.0)
        emb = jnp.dot(maskf, em.reshape(NG, 1), preferred_element_type=_f32)
        a = jnp.exp(e - emb)
        denom = lax.dot_general(maskf, a, (((0,), (0,)), ((), ())),
                                preferred_element_type=_f32)
        ab = a / (jnp.dot(maskf, denom, preferred_element_type=_f32) + 1e-16)
        rvec = lax.dot_general(maskf, out * ab, (((0,), (0,)), ((), ())),
                               preferred_element_type=_f32)
        q_star = jnp.concatenate([q, rvec], axis=1)
    ge = jnp.dot(q_star, w1t_ref[...], preferred_element_type=_f32) + b1_ref[...]
    ge_ref[...] = ge
    pred_ref[...] = jnp.dot(jnp.maximum(ge, 0.0), w2t_ref[...],
                            preferred_element_type=_f32) + b2_ref[...]


def _init_proj(x, w0t, b0r):
    return pl.pallas_call(
        _init_body,
        out_shape=jax.ShapeDtypeStruct((N, DIM), _f32),
    )(x, w0t, b0r)


def _edge_msgs(ea, xj, we1t, be1r, we2t, be2r):
    grid = (N_TILES,)
    return pl.pallas_call(
        _edge_body,
        grid=grid,
        in_specs=[
            pl.BlockSpec((EB, 3), lambda i: (i, 0)),
            pl.BlockSpec((EB, DIM), lambda i: (i, 0)),
            pl.BlockSpec((3, F), lambda i: (0, 0)),
            pl.BlockSpec((1, F), lambda i: (0, 0)),
            pl.BlockSpec((F, DIM * DIM), lambda i: (0, 0)),
            pl.BlockSpec((1, DIM * DIM), lambda i: (0, 0)),
        ],
        out_specs=pl.BlockSpec((EB, DIM), lambda i: (i, 0)),
        out_shape=jax.ShapeDtypeStruct((E, DIM), _f32),
    )(ea, xj, we1t, be1r, we2t, be2r)


def _node_update(agg, cnt, out, h, wroot, bconvr, wiht, bihr, whht, bhhr):
    return pl.pallas_call(
        _node_body,
        out_shape=jax.ShapeDtypeStruct((N, DIM), _f32),
    )(agg, cnt, out, h, wroot, bconvr, wiht, bihr, whht, bhhr)


def _set2set(out, batch2, wihst, bihsr, whhst, bhhsr, w1t, b1r, w2t, b2r):
    return pl.pallas_call(
        _set2set_body,
        out_shape=(jax.ShapeDtypeStruct((NG, 2 * DIM), _f32),
                   jax.ShapeDtypeStruct((NG, 1), _f32)),
    )(out, batch2, wihst, bihsr, whhst, bhhsr, w1t, b1r, w2t, b2r)


def _gather(table, src):
    return table[src]


def _scatter_sum(msg, dst):
    return jax.ops.segment_sum(msg, dst, num_segments=N)


def kernel(x, edge_index, edge_attr, batch, W0, b0, We1, be1, We2, be2,
           Wroot, bconv, Wih, Whh, bih, bhh, Wih_s, Whh_s, bih_s, bhh_s,
           W1, b1, W2, b2):
    src = edge_index[0]
    dst = edge_index[1]
    w0t = W0.T
    we1t = We1.T
    we2t = We2.T
    wiht = Wih.T
    whht = Whh.T
    wihst = Wih_s.T
    whhst = Whh_s.T
    w1t = W1.T
    w2t = W2.T
    b0r = b0.reshape(1, -1)
    be1r = be1.reshape(1, -1)
    be2r = be2.reshape(1, -1)
    bconvr = bconv.reshape(1, -1)
    bihr = bih.reshape(1, -1)
    bhhr = bhh.reshape(1, -1)
    bihsr = bih_s.reshape(1, -1)
    bhhsr = bhh_s.reshape(1, -1)
    b1r = b1.reshape(1, -1)
    b2r = b2.reshape(1, -1)
    batch2 = batch.reshape(N, 1)

    out = _init_proj(x, w0t, b0r)
    h = out
    cnt = _scatter_sum(jnp.ones((E, 1), _f32), dst)
    for _ in range(3):
        xj = _gather(out, src)
        msg = _edge_msgs(edge_attr, xj, we1t, be1r, we2t, be2r)
        agg = _scatter_sum(msg, dst)
        h = _node_update(agg, cnt, out, h, Wroot, bconvr, wiht, bihr, whht, bhhr)
        out = h
    ge, pred = _set2set(out, batch2, wihst, bihsr, whhst, bhhsr, w1t, b1r, w2t, b2r)
    return pred.reshape(-1), ge


# speedup vs baseline: 1.3506x; 1.3506x over previous
import functools

import jax
import jax.numpy as jnp
from jax import lax
from jax.experimental import pallas as pl
from jax.experimental.pallas import tpu as pltpu
from jax.experimental.pallas import tpu_sc as plsc

N = 10000
E = 160000
F = 128
DIM = 32
NG = 128
EB = 640
N_TILES = E // EB

NC = 2
NS = 16
NW = NC * NS
CK = 128
NCH = E // CK
OCT = 8
NOCT = NCH // OCT
N_PAD = 10240
NPT = N_PAD // NS

_f32 = jnp.float32
_HI = lax.Precision.HIGHEST


def _init_body(x_ref, w_ref, b_ref, o_ref):
    o_ref[...] = jnp.maximum(
        jnp.dot(x_ref[...], w_ref[...], preferred_element_type=_f32) + b_ref[...], 0.0)


def _edge_body(ea_ref, xj_ref, we1t_ref, be1_ref, we2t_ref, be2_ref, msg_ref):
    rh = jnp.maximum(
        jnp.dot(ea_ref[...], we1t_ref[...], preferred_element_type=_f32) + be1_ref[...], 0.0)
    p = jnp.dot(rh, we2t_ref[...], preferred_element_type=_f32) + be2_ref[...]
    xj = xj_ref[...]
    p3 = p.reshape(EB, DIM, DIM)
    msg_ref[...] = jnp.sum(p3 * xj[:, :, None], axis=1)


def _node_body(agg_ref, cnt_ref, out_ref, h_ref, wroot_ref, bconv_ref,
               wiht_ref, bih_ref, whht_ref, bhh_ref, hnew_ref):
    agg = agg_ref[0, 0:N, :] + agg_ref[1, 0:N, :]
    cnt = jnp.maximum(cnt_ref[0, 0:N, 0:1] + cnt_ref[1, 0:N, 0:1], 1.0)
    out = out_ref[...]
    h = h_ref[...]
    m = jnp.maximum(
        agg / cnt
        + jnp.dot(out, wroot_ref[...], preferred_element_type=_f32)
        + bconv_ref[...], 0.0)
    gi = jnp.dot(m, wiht_ref[...], preferred_element_type=_f32) + bih_ref[...]
    gh = jnp.dot(h, whht_ref[...], preferred_element_type=_f32) + bhh_ref[...]
    r = jax.nn.sigmoid(gi[:, 0:DIM] + gh[:, 0:DIM])
    z = jax.nn.sigmoid(gi[:, DIM:2 * DIM] + gh[:, DIM:2 * DIM])
    ng = jnp.tanh(gi[:, 2 * DIM:3 * DIM] + r * gh[:, 2 * DIM:3 * DIM])
    hnew_ref[...] = (1.0 - z) * ng + z * h


def _set2set_body(out_ref, batch_ref, wihst_ref, bihs_ref, whhst_ref, bhhs_ref,
                  w1t_ref, b1_ref, w2t_ref, b2_ref, ge_ref, pred_ref):
    out = out_ref[...]
    mask = batch_ref[...] == lax.broadcasted_iota(jnp.int32, (1, NG), 1)
    maskf = mask.astype(_f32)
    q_star = jnp.zeros((NG, 2 * DIM), dtype=_f32)
    hs = jnp.zeros((NG, DIM), dtype=_f32)
    cs = jnp.zeros((NG, DIM), dtype=_f32)
    for _ in range(3):
        gates = (jnp.dot(q_star, wihst_ref[...], preferred_element_type=_f32) + bihs_ref[...]
                 + jnp.dot(hs, whhst_ref[...], preferred_element_type=_f32) + bhhs_ref[...])
        gi = gates[:, 0:DIM]
        gf = gates[:, DIM:2 * DIM]
        gg = gates[:, 2 * DIM:3 * DIM]
        go = gates[:, 3 * DIM:4 * DIM]
        cs = jax.nn.sigmoid(gf) * cs + jax.nn.sigmoid(gi) * jnp.tanh(gg)
        hs = jax.nn.sigmoid(go) * jnp.tanh(cs)
        q = hs
        qb = jnp.dot(maskf, q, preferred_element_type=_f32, precision=_HI)
        e = jnp.sum(out * qb, axis=1, keepdims=True)
        em = jnp.max(jnp.where(mask, e, -1e30), axis=0, keepdims=True)
        em = jnp.where(em > -1e29, em, 0.0)
        emb = jnp.dot(maskf, em.reshape(NG, 1), preferred_element_type=_f32,
                      precision=_HI)
        a = jnp.exp(e - emb)
        denom = lax.dot_general(maskf, a, (((0,), (0,)), ((), ())),
                                preferred_element_type=_f32, precision=_HI)
        ab = a / (jnp.dot(maskf, denom, preferred_element_type=_f32,
                          precision=_HI) + 1e-16)
        rvec = lax.dot_general(maskf, out * ab, (((0,), (0,)), ((), ())),
                               preferred_element_type=_f32, precision=_HI)
        q_star = jnp.concatenate([q, rvec], axis=1)
    ge = jnp.dot(q_star, w1t_ref[...], preferred_element_type=_f32) + b1_ref[...]
    ge_ref[...] = ge
    pred_ref[...] = jnp.dot(jnp.maximum(ge, 0.0), w2t_ref[...],
                            preferred_element_type=_f32) + b2_ref[...]


def _init_proj(x, w0t, b0r):
    return pl.pallas_call(
        _init_body,
        out_shape=jax.ShapeDtypeStruct((N, DIM), _f32),
    )(x, w0t, b0r)


def _edge_msgs(ea, xj, we1t, be1r, we2t, be2r):
    grid = (N_TILES,)
    return pl.pallas_call(
        _edge_body,
        grid=grid,
        in_specs=[
            pl.BlockSpec((EB, 3), lambda i: (i, 0)),
            pl.BlockSpec((EB, DIM), lambda i: (i, 0)),
            pl.BlockSpec((3, F), lambda i: (0, 0)),
            pl.BlockSpec((1, F), lambda i: (0, 0)),
            pl.BlockSpec((F, DIM * DIM), lambda i: (0, 0)),
            pl.BlockSpec((1, DIM * DIM), lambda i: (0, 0)),
        ],
        out_specs=pl.BlockSpec((EB, DIM), lambda i: (i, 0)),
        out_shape=jax.ShapeDtypeStruct((E, DIM), _f32),
        compiler_params=pltpu.CompilerParams(vmem_limit_bytes=100 * 2**20),
    )(ea, xj, we1t, be1r, we2t, be2r)


def _node_update(aggp, cntp, out, h, wroot, bconvr, wiht, bihr, whht, bhhr):
    return pl.pallas_call(
        _node_body,
        out_shape=jax.ShapeDtypeStruct((N, DIM), _f32),
    )(aggp, cntp, out, h, wroot, bconvr, wiht, bihr, whht, bhhr)


def _set2set(out, batch2, wihst, bihsr, whhst, bhhsr, w1t, b1r, w2t, b2r):
    return pl.pallas_call(
        _set2set_body,
        out_shape=(jax.ShapeDtypeStruct((NG, 2 * DIM), _f32),
                   jax.ShapeDtypeStruct((NG, 1), _f32)),
        compiler_params=pltpu.CompilerParams(vmem_limit_bytes=100 * 2**20),
    )(out, batch2, wihst, bihsr, whhst, bhhsr, w1t, b1r, w2t, b2r)


def _gather_body(table, idx2, out, idx_v, rows_v, semg):
    wid = lax.axis_index("s") * NC + lax.axis_index("c")

    def do_batch(ch0, nch):
        pltpu.sync_copy(idx2.at[pl.ds(ch0, nch)], idx_v.at[pl.ds(0, nch)])
        descs = [pltpu.async_copy(table.at[idx_v.at[b]],
                                  rows_v.at[pl.ds(b * CK, CK)], semg)
                 for b in range(nch)]
        for d in descs:
            d.wait()
        pltpu.sync_copy(rows_v.at[pl.ds(0, nch * CK)],
                        out.at[pl.ds(ch0 * CK, nch * CK)])

    def body(g, _):
        o = g * NW + wid

        @pl.when(o < NOCT)
        def _():
            do_batch(o * OCT, OCT)

        return ()

    lax.fori_loop(0, (NOCT + NW - 1) // NW, body, ())

    @pl.when(wid == 0)
    def _tail():
        do_batch(NOCT * OCT, NCH - NOCT * OCT)


def _sc_gather(table, idx2):
    mesh = plsc.VectorSubcoreMesh(core_axis_name="c", subcore_axis_name="s")
    f = functools.partial(
        pl.kernel,
        out_type=jax.ShapeDtypeStruct((E, DIM), _f32),
        mesh=mesh,
        compiler_params=pltpu.CompilerParams(use_tc_tiling_on_sc=False),
        scratch_types=[
            pltpu.VMEM((OCT, CK), jnp.int32),
            pltpu.VMEM((OCT * CK, DIM), _f32),
            pltpu.SemaphoreType.DMA,
        ],
    )(_gather_body)
    return f(table, idx2)


def _scatter_body(msg, idx2, zeros, out, idx_v, msg_v, agg_sh):
    cid = lax.axis_index("c")
    sid = lax.axis_index("s")
    wid = sid * NC + cid
    pltpu.sync_copy(zeros.at[pl.ds(sid * NPT, NPT)],
                    agg_sh.at[pl.ds(sid * NPT, NPT)])
    plsc.subcore_barrier()

    def do_batch(ch0, nch):
        pltpu.sync_copy(idx2.at[pl.ds(ch0, nch)], idx_v.at[pl.ds(0, nch)])
        pltpu.sync_copy(msg.at[pl.ds(ch0 * CK, nch * CK)],
                        msg_v.at[pl.ds(0, nch * CK)])
        for b in range(nch):
            pltpu.sync_copy(msg_v.at[pl.ds(b * CK, CK)],
                            agg_sh.at[idx_v.at[b]], add=True)

    def body(g, _):
        o = g * NW + wid

        @pl.when(o < NOCT)
        def _():
            do_batch(o * OCT, OCT)

        return ()

    lax.fori_loop(0, (NOCT + NW - 1) // NW, body, ())

    @pl.when(wid == 0)
    def _tail():
        do_batch(NOCT * OCT, NCH - NOCT * OCT)

    plsc.subcore_barrier()
    pltpu.sync_copy(agg_sh.at[pl.ds(sid * NPT, NPT)],
                    out.at[cid, pl.ds(sid * NPT, NPT)])


def _sc_scatter(msg, idx2, zeros):
    mesh = plsc.VectorSubcoreMesh(core_axis_name="c", subcore_axis_name="s")
    f = functools.partial(
        pl.kernel,
        out_type=jax.ShapeDtypeStruct((NC, N_PAD, DIM), _f32),
        mesh=mesh,
        compiler_params=pltpu.CompilerParams(use_tc_tiling_on_sc=False),
        scratch_types=[
            pltpu.VMEM((OCT, CK), jnp.int32),
            pltpu.VMEM((OCT * CK, DIM), _f32),
            pltpu.VMEM_SHARED((N_PAD, DIM), _f32),
        ],
    )(_scatter_body)
    return f(msg, idx2, zeros)


def kernel(x, edge_index, edge_attr, batch, W0, b0, We1, be1, We2, be2,
           Wroot, bconv, Wih, Whh, bih, bhh, Wih_s, Whh_s, bih_s, bhh_s,
           W1, b1, W2, b2):
    src = edge_index[0]
    dst = edge_index[1]
    w0t = W0.T
    we1t = We1.T
    we2t = We2.T
    wiht = Wih.T
    whht = Whh.T
    wihst = Wih_s.T
    whhst = Whh_s.T
    w1t = W1.T
    w2t = W2.T
    b0r = b0.reshape(1, -1)
    be1r = be1.reshape(1, -1)
    be2r = be2.reshape(1, -1)
    bconvr = bconv.reshape(1, -1)
    bihr = bih.reshape(1, -1)
    bhhr = bhh.reshape(1, -1)
    bihsr = bih_s.reshape(1, -1)
    bhhsr = bhh_s.reshape(1, -1)
    b1r = b1.reshape(1, -1)
    b2r = b2.reshape(1, -1)
    batch2 = batch.reshape(N, 1)

    src2 = src.reshape(NCH, CK)
    dst2 = dst.reshape(NCH, CK)
    zero_nd = jnp.zeros((N_PAD, DIM), _f32)
    ones_e = jnp.ones((E, DIM), _f32)

    out = _init_proj(x, w0t, b0r)
    h = out
    cntp = _sc_scatter(ones_e, dst2, zero_nd)
    for _ in range(3):
        xj = _sc_gather(out, src2)
        msg = _edge_msgs(edge_attr, xj, we1t, be1r, we2t, be2r)
        aggp = _sc_scatter(msg, dst2, zero_nd)
        h = _node_update(aggp, cntp, out, h, Wroot, bconvr, wiht, bihr, whht, bhhr)
        out = h
    ge, pred = _set2set(out, batch2, wihst, bihsr, whhst, bhhsr, w1t, b1r, w2t, b2r)
    return pred.reshape(-1), ge


# speedup vs baseline: 1.3560x; 1.0040x over previous
import functools

import jax
import jax.numpy as jnp
from jax import lax
from jax.experimental import pallas as pl
from jax.experimental.pallas import tpu as pltpu
from jax.experimental.pallas import tpu_sc as plsc

N = 10000
E = 160000
F = 128
DIM = 32
NG = 128
EB = 640
N_TILES = E // EB

NC = 2
NS = 16
NW = NC * NS
CK = 128
NCH = E // CK
OCT = 8
NOCT = NCH // OCT
N_PAD = 10240
NPT = N_PAD // NS

_f32 = jnp.float32
_HI = lax.Precision.HIGHEST


def _init_body(x_ref, w_ref, b_ref, o_ref):
    o_ref[...] = jnp.maximum(
        jnp.dot(x_ref[...], w_ref[...], preferred_element_type=_f32) + b_ref[...], 0.0)


def _edge_body(ea_ref, xj_ref, we1t_ref, be1_ref, we2t_ref, be2_ref, msg_ref):
    rh = jnp.maximum(
        jnp.dot(ea_ref[...], we1t_ref[...], preferred_element_type=_f32) + be1_ref[...], 0.0)
    p = jnp.dot(rh, we2t_ref[...], preferred_element_type=_f32) + be2_ref[...]
    xj = xj_ref[...]
    z = p * jnp.repeat(xj, DIM, axis=1)
    z = z[:, 0:512] + z[:, 512:1024]
    z = z[:, 0:256] + z[:, 256:512]
    z = z[:, 0:128] + z[:, 128:256]
    z = z[:, 0:64] + z[:, 64:128]
    msg_ref[...] = z[:, 0:32] + z[:, 32:64]


def _node_body(agg_ref, cnt_ref, out_ref, h_ref, wroot_ref, bconv_ref,
               wiht_ref, bih_ref, whht_ref, bhh_ref, hnew_ref):
    agg = agg_ref[0, 0:N, :] + agg_ref[1, 0:N, :]
    cnt = jnp.maximum(cnt_ref[0, 0:N, 0:1] + cnt_ref[1, 0:N, 0:1], 1.0)
    out = out_ref[...]
    h = h_ref[...]
    m = jnp.maximum(
        agg / cnt
        + jnp.dot(out, wroot_ref[...], preferred_element_type=_f32)
        + bconv_ref[...], 0.0)
    gi = jnp.dot(m, wiht_ref[...], preferred_element_type=_f32) + bih_ref[...]
    gh = jnp.dot(h, whht_ref[...], preferred_element_type=_f32) + bhh_ref[...]
    r = jax.nn.sigmoid(gi[:, 0:DIM] + gh[:, 0:DIM])
    z = jax.nn.sigmoid(gi[:, DIM:2 * DIM] + gh[:, DIM:2 * DIM])
    ng = jnp.tanh(gi[:, 2 * DIM:3 * DIM] + r * gh[:, 2 * DIM:3 * DIM])
    hnew_ref[...] = (1.0 - z) * ng + z * h


def _set2set_body(out_ref, batch_ref, wihst_ref, bihs_ref, whhst_ref, bhhs_ref,
                  w1t_ref, b1_ref, w2t_ref, b2_ref, ge_ref, pred_ref):
    out = out_ref[...]
    mask = batch_ref[...] == lax.broadcasted_iota(jnp.int32, (1, NG), 1)
    maskf = mask.astype(_f32)
    q_star = jnp.zeros((NG, 2 * DIM), dtype=_f32)
    hs = jnp.zeros((NG, DIM), dtype=_f32)
    cs = jnp.zeros((NG, DIM), dtype=_f32)
    for _ in range(3):
        gates = (jnp.dot(q_star, wihst_ref[...], preferred_element_type=_f32) + bihs_ref[...]
                 + jnp.dot(hs, whhst_ref[...], preferred_element_type=_f32) + bhhs_ref[...])
        gi = gates[:, 0:DIM]
        gf = gates[:, DIM:2 * DIM]
        gg = gates[:, 2 * DIM:3 * DIM]
        go = gates[:, 3 * DIM:4 * DIM]
        cs = jax.nn.sigmoid(gf) * cs + jax.nn.sigmoid(gi) * jnp.tanh(gg)
        hs = jax.nn.sigmoid(go) * jnp.tanh(cs)
        q = hs
        qb = jnp.dot(maskf, q, preferred_element_type=_f32, precision=_HI)
        e = jnp.sum(out * qb, axis=1, keepdims=True)
        em = jnp.max(jnp.where(mask, e, -1e30), axis=0, keepdims=True)
        em = jnp.where(em > -1e29, em, 0.0)
        emb = jnp.dot(maskf, em.reshape(NG, 1), preferred_element_type=_f32,
                      precision=_HI)
        a = jnp.exp(e - emb)
        denom = lax.dot_general(maskf, a, (((0,), (0,)), ((), ())),
                                preferred_element_type=_f32, precision=_HI)
        ab = a / (jnp.dot(maskf, denom, preferred_element_type=_f32,
                          precision=_HI) + 1e-16)
        rvec = lax.dot_general(maskf, out * ab, (((0,), (0,)), ((), ())),
                               preferred_element_type=_f32, precision=_HI)
        q_star = jnp.concatenate([q, rvec], axis=1)
    ge = jnp.dot(q_star, w1t_ref[...], preferred_element_type=_f32) + b1_ref[...]
    ge_ref[...] = ge
    pred_ref[...] = jnp.dot(jnp.maximum(ge, 0.0), w2t_ref[...],
                            preferred_element_type=_f32) + b2_ref[...]


def _init_proj(x, w0t, b0r):
    return pl.pallas_call(
        _init_body,
        out_shape=jax.ShapeDtypeStruct((N, DIM), _f32),
    )(x, w0t, b0r)


def _edge_msgs(ea, xj, we1t, be1r, we2t, be2r):
    grid = (N_TILES,)
    return pl.pallas_call(
        _edge_body,
        grid=grid,
        in_specs=[
            pl.BlockSpec((EB, 3), lambda i: (i, 0)),
            pl.BlockSpec((EB, DIM), lambda i: (i, 0)),
            pl.BlockSpec((3, F), lambda i: (0, 0)),
            pl.BlockSpec((1, F), lambda i: (0, 0)),
            pl.BlockSpec((F, DIM * DIM), lambda i: (0, 0)),
            pl.BlockSpec((1, DIM * DIM), lambda i: (0, 0)),
        ],
        out_specs=pl.BlockSpec((EB, DIM), lambda i: (i, 0)),
        out_shape=jax.ShapeDtypeStruct((E, DIM), _f32),
        compiler_params=pltpu.CompilerParams(vmem_limit_bytes=100 * 2**20),
    )(ea, xj, we1t, be1r, we2t, be2r)


def _node_update(aggp, cntp, out, h, wroot, bconvr, wiht, bihr, whht, bhhr):
    return pl.pallas_call(
        _node_body,
        out_shape=jax.ShapeDtypeStruct((N, DIM), _f32),
    )(aggp, cntp, out, h, wroot, bconvr, wiht, bihr, whht, bhhr)


def _set2set(out, batch2, wihst, bihsr, whhst, bhhsr, w1t, b1r, w2t, b2r):
    return pl.pallas_call(
        _set2set_body,
        out_shape=(jax.ShapeDtypeStruct((NG, 2 * DIM), _f32),
                   jax.ShapeDtypeStruct((NG, 1), _f32)),
        compiler_params=pltpu.CompilerParams(vmem_limit_bytes=100 * 2**20),
    )(out, batch2, wihst, bihsr, whhst, bhhsr, w1t, b1r, w2t, b2r)


def _gather_body(table, idx2, out, idx_v, rows_v, semg):
    wid = lax.axis_index("s") * NC + lax.axis_index("c")

    def do_batch(ch0, nch):
        pltpu.sync_copy(idx2.at[pl.ds(ch0, nch)], idx_v.at[pl.ds(0, nch)])
        descs = [pltpu.async_copy(table.at[idx_v.at[b]],
                                  rows_v.at[pl.ds(b * CK, CK)], semg)
                 for b in range(nch)]
        for d in descs:
            d.wait()
        pltpu.sync_copy(rows_v.at[pl.ds(0, nch * CK)],
                        out.at[pl.ds(ch0 * CK, nch * CK)])

    def body(g, _):
        o = g * NW + wid

        @pl.when(o < NOCT)
        def _():
            do_batch(o * OCT, OCT)

        return ()

    lax.fori_loop(0, (NOCT + NW - 1) // NW, body, ())

    @pl.when(wid == 0)
    def _tail():
        do_batch(NOCT * OCT, NCH - NOCT * OCT)


def _sc_gather(table, idx2):
    mesh = plsc.VectorSubcoreMesh(core_axis_name="c", subcore_axis_name="s")
    f = functools.partial(
        pl.kernel,
        out_type=jax.ShapeDtypeStruct((E, DIM), _f32),
        mesh=mesh,
        compiler_params=pltpu.CompilerParams(use_tc_tiling_on_sc=False),
        scratch_types=[
            pltpu.VMEM((OCT, CK), jnp.int32),
            pltpu.VMEM((OCT * CK, DIM), _f32),
            pltpu.SemaphoreType.DMA,
        ],
    )(_gather_body)
    return f(table, idx2)


def _scatter_body(msg, idx2, zeros, out, idx_v, msg_v, agg_sh):
    cid = lax.axis_index("c")
    sid = lax.axis_index("s")
    wid = sid * NC + cid
    pltpu.sync_copy(zeros.at[pl.ds(sid * NPT, NPT)],
                    agg_sh.at[pl.ds(sid * NPT, NPT)])
    plsc.subcore_barrier()

    def do_batch(ch0, nch):
        pltpu.sync_copy(idx2.at[pl.ds(ch0, nch)], idx_v.at[pl.ds(0, nch)])
        pltpu.sync_copy(msg.at[pl.ds(ch0 * CK, nch * CK)],
                        msg_v.at[pl.ds(0, nch * CK)])
        for b in range(nch):
            pltpu.sync_copy(msg_v.at[pl.ds(b * CK, CK)],
                            agg_sh.at[idx_v.at[b]], add=True)

    def body(g, _):
        o = g * NW + wid

        @pl.when(o < NOCT)
        def _():
            do_batch(o * OCT, OCT)

        return ()

    lax.fori_loop(0, (NOCT + NW - 1) // NW, body, ())

    @pl.when(wid == 0)
    def _tail():
        do_batch(NOCT * OCT, NCH - NOCT * OCT)

    plsc.subcore_barrier()
    pltpu.sync_copy(agg_sh.at[pl.ds(sid * NPT, NPT)],
                    out.at[cid, pl.ds(sid * NPT, NPT)])


def _sc_scatter(msg, idx2, zeros):
    mesh = plsc.VectorSubcoreMesh(core_axis_name="c", subcore_axis_name="s")
    f = functools.partial(
        pl.kernel,
        out_type=jax.ShapeDtypeStruct((NC, N_PAD, DIM), _f32),
        mesh=mesh,
        compiler_params=pltpu.CompilerParams(use_tc_tiling_on_sc=False),
        scratch_types=[
            pltpu.VMEM((OCT, CK), jnp.int32),
            pltpu.VMEM((OCT * CK, DIM), _f32),
            pltpu.VMEM_SHARED((N_PAD, DIM), _f32),
        ],
    )(_scatter_body)
    return f(msg, idx2, zeros)


def kernel(x, edge_index, edge_attr, batch, W0, b0, We1, be1, We2, be2,
           Wroot, bconv, Wih, Whh, bih, bhh, Wih_s, Whh_s, bih_s, bhh_s,
           W1, b1, W2, b2):
    src = edge_index[0]
    dst = edge_index[1]
    w0t = W0.T
    we1t = We1.T
    we2t = We2.T
    wiht = Wih.T
    whht = Whh.T
    wihst = Wih_s.T
    whhst = Whh_s.T
    w1t = W1.T
    w2t = W2.T
    b0r = b0.reshape(1, -1)
    be1r = be1.reshape(1, -1)
    be2r = be2.reshape(1, -1)
    bconvr = bconv.reshape(1, -1)
    bihr = bih.reshape(1, -1)
    bhhr = bhh.reshape(1, -1)
    bihsr = bih_s.reshape(1, -1)
    bhhsr = bhh_s.reshape(1, -1)
    b1r = b1.reshape(1, -1)
    b2r = b2.reshape(1, -1)
    batch2 = batch.reshape(N, 1)

    src2 = src.reshape(NCH, CK)
    dst2 = dst.reshape(NCH, CK)
    zero_nd = jnp.zeros((N_PAD, DIM), _f32)
    ones_e = jnp.ones((E, DIM), _f32)

    out = _init_proj(x, w0t, b0r)
    h = out
    cntp = _sc_scatter(ones_e, dst2, zero_nd)
    for _ in range(3):
        xj = _sc_gather(out, src2)
        msg = _edge_msgs(edge_attr, xj, we1t, be1r, we2t, be2r)
        aggp = _sc_scatter(msg, dst2, zero_nd)
        h = _node_update(aggp, cntp, out, h, Wroot, bconvr, wiht, bihr, whht, bhhr)
        out = h
    ge, pred = _set2set(out, batch2, wihst, bihsr, whhst, bhhsr, w1t, b1r, w2t, b2r)
    return pred.reshape(-1), ge


# speedup vs baseline: 2.0465x; 1.5093x over previous
import functools

import jax
import jax.numpy as jnp
from jax import lax
from jax.experimental import pallas as pl
from jax.experimental.pallas import tpu as pltpu
from jax.experimental.pallas import tpu_sc as plsc

N = 10000
E = 160000
F = 128
DIM = 32
NG = 128
EB = 640
N_TILES = E // EB

NC = 2
NS = 16
NW = NC * NS
CK = 128
NCH = E // CK
OCT = 8
NOCT = NCH // OCT
N_PAD = 10240
NPT = N_PAD // NS

_f32 = jnp.float32
_HI = lax.Precision.HIGHEST


def _init_body(x_ref, w_ref, b_ref, o_ref):
    o_ref[...] = jnp.maximum(
        jnp.dot(x_ref[...], w_ref[...], preferred_element_type=_f32) + b_ref[...], 0.0)


def _edge_body(ea_ref, xj_ref, we1t_ref, be1_ref, we2t_ref, be2_ref, msg_ref):
    rh = jnp.maximum(
        jnp.dot(ea_ref[...], we1t_ref[...], preferred_element_type=_f32) + be1_ref[...], 0.0)
    p = jnp.dot(rh, we2t_ref[...], preferred_element_type=_f32) + be2_ref[...]
    xj = xj_ref[...]
    acc = jnp.zeros((EB, 4 * DIM), dtype=_f32)
    for j in range(8):
        w = jnp.concatenate(
            [jnp.broadcast_to(xj[:, 4 * j + c:4 * j + c + 1], (EB, DIM))
             for c in range(4)], axis=1)
        acc = acc + p[:, 128 * j:128 * (j + 1)] * w
    r = acc[:, 0:64] + acc[:, 64:128]
    msg_ref[...] = r[:, 0:32] + r[:, 32:64]


def _node_body(agg_ref, cnt_ref, out_ref, h_ref, wroot_ref, bconv_ref,
               wiht_ref, bih_ref, whht_ref, bhh_ref, hnew_ref):
    agg = agg_ref[0, 0:N, :] + agg_ref[1, 0:N, :]
    cnt = jnp.maximum(cnt_ref[0, 0:N, 0:1] + cnt_ref[1, 0:N, 0:1], 1.0)
    out = out_ref[...]
    h = h_ref[...]
    m = jnp.maximum(
        agg / cnt
        + jnp.dot(out, wroot_ref[...], preferred_element_type=_f32)
        + bconv_ref[...], 0.0)
    gi = jnp.dot(m, wiht_ref[...], preferred_element_type=_f32) + bih_ref[...]
    gh = jnp.dot(h, whht_ref[...], preferred_element_type=_f32) + bhh_ref[...]
    r = jax.nn.sigmoid(gi[:, 0:DIM] + gh[:, 0:DIM])
    z = jax.nn.sigmoid(gi[:, DIM:2 * DIM] + gh[:, DIM:2 * DIM])
    ng = jnp.tanh(gi[:, 2 * DIM:3 * DIM] + r * gh[:, 2 * DIM:3 * DIM])
    hnew_ref[...] = (1.0 - z) * ng + z * h


def _set2set_body(out_ref, batch_ref, wihst_ref, bihs_ref, whhst_ref, bhhs_ref,
                  w1t_ref, b1_ref, w2t_ref, b2_ref, ge_ref, pred_ref):
    out = out_ref[...]
    mask = batch_ref[...] == lax.broadcasted_iota(jnp.int32, (1, NG), 1)
    maskf = mask.astype(_f32)
    q_star = jnp.zeros((NG, 2 * DIM), dtype=_f32)
    hs = jnp.zeros((NG, DIM), dtype=_f32)
    cs = jnp.zeros((NG, DIM), dtype=_f32)
    for _ in range(3):
        gates = (jnp.dot(q_star, wihst_ref[...], preferred_element_type=_f32) + bihs_ref[...]
                 + jnp.dot(hs, whhst_ref[...], preferred_element_type=_f32) + bhhs_ref[...])
        gi = gates[:, 0:DIM]
        gf = gates[:, DIM:2 * DIM]
        gg = gates[:, 2 * DIM:3 * DIM]
        go = gates[:, 3 * DIM:4 * DIM]
        cs = jax.nn.sigmoid(gf) * cs + jax.nn.sigmoid(gi) * jnp.tanh(gg)
        hs = jax.nn.sigmoid(go) * jnp.tanh(cs)
        q = hs
        qb = jnp.dot(maskf, q, preferred_element_type=_f32, precision=_HI)
        e = jnp.sum(out * qb, axis=1, keepdims=True)
        em = jnp.max(jnp.where(mask, e, -1e30), axis=0, keepdims=True)
        em = jnp.where(em > -1e29, em, 0.0)
        emb = jnp.dot(maskf, em.reshape(NG, 1), preferred_element_type=_f32,
                      precision=_HI)
        a = jnp.exp(e - emb)
        denom = lax.dot_general(maskf, a, (((0,), (0,)), ((), ())),
                                preferred_element_type=_f32, precision=_HI)
        ab = a / (jnp.dot(maskf, denom, preferred_element_type=_f32,
                          precision=_HI) + 1e-16)
        rvec = lax.dot_general(maskf, out * ab, (((0,), (0,)), ((), ())),
                               preferred_element_type=_f32, precision=_HI)
        q_star = jnp.concatenate([q, rvec], axis=1)
    ge = jnp.dot(q_star, w1t_ref[...], preferred_element_type=_f32) + b1_ref[...]
    ge_ref[...] = ge
    pred_ref[...] = jnp.dot(jnp.maximum(ge, 0.0), w2t_ref[...],
                            preferred_element_type=_f32) + b2_ref[...]


def _init_proj(x, w0t, b0r):
    return pl.pallas_call(
        _init_body,
        out_shape=jax.ShapeDtypeStruct((N, DIM), _f32),
    )(x, w0t, b0r)


def _edge_msgs(ea, xj, we1t, be1r, we2t, be2r):
    grid = (N_TILES,)
    return pl.pallas_call(
        _edge_body,
        grid=grid,
        in_specs=[
            pl.BlockSpec((EB, 3), lambda i: (i, 0)),
            pl.BlockSpec((EB, DIM), lambda i: (i, 0)),
            pl.BlockSpec((3, F), lambda i: (0, 0)),
            pl.BlockSpec((1, F), lambda i: (0, 0)),
            pl.BlockSpec((F, DIM * DIM), lambda i: (0, 0)),
            pl.BlockSpec((1, DIM * DIM), lambda i: (0, 0)),
        ],
        out_specs=pl.BlockSpec((EB, DIM), lambda i: (i, 0)),
        out_shape=jax.ShapeDtypeStruct((E, DIM), _f32),
        compiler_params=pltpu.CompilerParams(vmem_limit_bytes=100 * 2**20),
    )(ea, xj, we1t, be1r, we2t, be2r)


def _node_update(aggp, cntp, out, h, wroot, bconvr, wiht, bihr, whht, bhhr):
    return pl.pallas_call(
        _node_body,
        out_shape=jax.ShapeDtypeStruct((N, DIM), _f32),
    )(aggp, cntp, out, h, wroot, bconvr, wiht, bihr, whht, bhhr)


def _set2set(out, batch2, wihst, bihsr, whhst, bhhsr, w1t, b1r, w2t, b2r):
    return pl.pallas_call(
        _set2set_body,
        out_shape=(jax.ShapeDtypeStruct((NG, 2 * DIM), _f32),
                   jax.ShapeDtypeStruct((NG, 1), _f32)),
        compiler_params=pltpu.CompilerParams(vmem_limit_bytes=100 * 2**20),
    )(out, batch2, wihst, bihsr, whhst, bhhsr, w1t, b1r, w2t, b2r)


def _gather_body(table, idx2, out, idx_v, rows_v, semg):
    wid = lax.axis_index("s") * NC + lax.axis_index("c")

    def do_batch(ch0, nch):
        pltpu.sync_copy(idx2.at[pl.ds(ch0, nch)], idx_v.at[pl.ds(0, nch)])
        descs = [pltpu.async_copy(table.at[idx_v.at[b]],
                                  rows_v.at[pl.ds(b * CK, CK)], semg)
                 for b in range(nch)]
        for d in descs:
            d.wait()
        pltpu.sync_copy(rows_v.at[pl.ds(0, nch * CK)],
                        out.at[pl.ds(ch0 * CK, nch * CK)])

    def body(g, _):
        o = g * NW + wid

        @pl.when(o < NOCT)
        def _():
            do_batch(o * OCT, OCT)

        return ()

    lax.fori_loop(0, (NOCT + NW - 1) // NW, body, ())

    @pl.when(wid == 0)
    def _tail():
        do_batch(NOCT * OCT, NCH - NOCT * OCT)


def _sc_gather(table, idx2):
    mesh = plsc.VectorSubcoreMesh(core_axis_name="c", subcore_axis_name="s")
    f = functools.partial(
        pl.kernel,
        out_type=jax.ShapeDtypeStruct((E, DIM), _f32),
        mesh=mesh,
        compiler_params=pltpu.CompilerParams(use_tc_tiling_on_sc=False),
        scratch_types=[
            pltpu.VMEM((OCT, CK), jnp.int32),
            pltpu.VMEM((OCT * CK, DIM), _f32),
            pltpu.SemaphoreType.DMA,
        ],
    )(_gather_body)
    return f(table, idx2)


def _scatter_body(msg, idx2, zeros, out, idx_v, msg_v, agg_sh):
    cid = lax.axis_index("c")
    sid = lax.axis_index("s")
    wid = sid * NC + cid
    pltpu.sync_copy(zeros.at[pl.ds(sid * NPT, NPT)],
                    agg_sh.at[pl.ds(sid * NPT, NPT)])
    plsc.subcore_barrier()

    def do_batch(ch0, nch):
        pltpu.sync_copy(idx2.at[pl.ds(ch0, nch)], idx_v.at[pl.ds(0, nch)])
        pltpu.sync_copy(msg.at[pl.ds(ch0 * CK, nch * CK)],
                        msg_v.at[pl.ds(0, nch * CK)])
        for b in range(nch):
            pltpu.sync_copy(msg_v.at[pl.ds(b * CK, CK)],
                            agg_sh.at[idx_v.at[b]], add=True)

    def body(g, _):
        o = g * NW + wid

        @pl.when(o < NOCT)
        def _():
            do_batch(o * OCT, OCT)

        return ()

    lax.fori_loop(0, (NOCT + NW - 1) // NW, body, ())

    @pl.when(wid == 0)
    def _tail():
        do_batch(NOCT * OCT, NCH - NOCT * OCT)

    plsc.subcore_barrier()
    pltpu.sync_copy(agg_sh.at[pl.ds(sid * NPT, NPT)],
                    out.at[cid, pl.ds(sid * NPT, NPT)])


def _sc_scatter(msg, idx2, zeros):
    mesh = plsc.VectorSubcoreMesh(core_axis_name="c", subcore_axis_name="s")
    f = functools.partial(
        pl.kernel,
        out_type=jax.ShapeDtypeStruct((NC, N_PAD, DIM), _f32),
        mesh=mesh,
        compiler_params=pltpu.CompilerParams(use_tc_tiling_on_sc=False),
        scratch_types=[
            pltpu.VMEM((OCT, CK), jnp.int32),
            pltpu.VMEM((OCT * CK, DIM), _f32),
            pltpu.VMEM_SHARED((N_PAD, DIM), _f32),
        ],
    )(_scatter_body)
    return f(msg, idx2, zeros)


def kernel(x, edge_index, edge_attr, batch, W0, b0, We1, be1, We2, be2,
           Wroot, bconv, Wih, Whh, bih, bhh, Wih_s, Whh_s, bih_s, bhh_s,
           W1, b1, W2, b2):
    src = edge_index[0]
    dst = edge_index[1]
    w0t = W0.T
    we1t = We1.T
    we2t = We2.T
    wiht = Wih.T
    whht = Whh.T
    wihst = Wih_s.T
    whhst = Whh_s.T
    w1t = W1.T
    w2t = W2.T
    b0r = b0.reshape(1, -1)
    be1r = be1.reshape(1, -1)
    be2r = be2.reshape(1, -1)
    bconvr = bconv.reshape(1, -1)
    bihr = bih.reshape(1, -1)
    bhhr = bhh.reshape(1, -1)
    bihsr = bih_s.reshape(1, -1)
    bhhsr = bhh_s.reshape(1, -1)
    b1r = b1.reshape(1, -1)
    b2r = b2.reshape(1, -1)
    batch2 = batch.reshape(N, 1)

    src2 = src.reshape(NCH, CK)
    dst2 = dst.reshape(NCH, CK)
    zero_nd = jnp.zeros((N_PAD, DIM), _f32)
    ones_e = jnp.ones((E, DIM), _f32)

    out = _init_proj(x, w0t, b0r)
    h = out
    cntp = _sc_scatter(ones_e, dst2, zero_nd)
    for _ in range(3):
        xj = _sc_gather(out, src2)
        msg = _edge_msgs(edge_attr, xj, we1t, be1r, we2t, be2r)
        aggp = _sc_scatter(msg, dst2, zero_nd)
        h = _node_update(aggp, cntp, out, h, Wroot, bconvr, wiht, bihr, whht, bhhr)
        out = h
    ge, pred = _set2set(out, batch2, wihst, bihsr, whhst, bhhsr, w1t, b1r, w2t, b2r)
    return pred.reshape(-1), ge


# speedup vs baseline: 2.0833x; 1.0180x over previous
import functools

import jax
import jax.numpy as jnp
from jax import lax
from jax.experimental import pallas as pl
from jax.experimental.pallas import tpu as pltpu
from jax.experimental.pallas import tpu_sc as plsc

N = 10000
E = 160000
F = 128
DIM = 32
NG = 128
EB = 1280
N_TILES = E // EB

NC = 2
NS = 16
NW = NC * NS
CK = 128
NCH = E // CK
OCT = 8
NOCT = NCH // OCT
N_PAD = 10240
NPT = N_PAD // NS

_f32 = jnp.float32
_HI = lax.Precision.HIGHEST


def _init_body(x_ref, w_ref, b_ref, o_ref):
    o_ref[...] = jnp.maximum(
        jnp.dot(x_ref[...], w_ref[...], preferred_element_type=_f32) + b_ref[...], 0.0)


def _edge_body(ea_ref, xj_ref, we1t_ref, be1_ref, we2t_ref, be2_ref, msg_ref):
    rh = jnp.maximum(
        jnp.dot(ea_ref[...], we1t_ref[...], preferred_element_type=_f32) + be1_ref[...], 0.0)
    p = jnp.dot(rh, we2t_ref[...], preferred_element_type=_f32) + be2_ref[...]
    xj = xj_ref[...]
    acc = jnp.zeros((EB, 4 * DIM), dtype=_f32)
    for j in range(8):
        w = jnp.concatenate(
            [jnp.broadcast_to(xj[:, 4 * j + c:4 * j + c + 1], (EB, DIM))
             for c in range(4)], axis=1)
        acc = acc + p[:, 128 * j:128 * (j + 1)] * w
    r = acc[:, 0:64] + acc[:, 64:128]
    msg_ref[...] = r[:, 0:32] + r[:, 32:64]


def _node_body(agg_ref, cnt_ref, out_ref, h_ref, wroot_ref, bconv_ref,
               wiht_ref, bih_ref, whht_ref, bhh_ref, hnew_ref):
    agg = agg_ref[0, 0:N, :] + agg_ref[1, 0:N, :]
    cnt = jnp.maximum(cnt_ref[0, 0:N, 0:1] + cnt_ref[1, 0:N, 0:1], 1.0)
    out = out_ref[...]
    h = h_ref[...]
    m = jnp.maximum(
        agg / cnt
        + jnp.dot(out, wroot_ref[...], preferred_element_type=_f32)
        + bconv_ref[...], 0.0)
    gi = jnp.dot(m, wiht_ref[...], preferred_element_type=_f32) + bih_ref[...]
    gh = jnp.dot(h, whht_ref[...], preferred_element_type=_f32) + bhh_ref[...]
    r = jax.nn.sigmoid(gi[:, 0:DIM] + gh[:, 0:DIM])
    z = jax.nn.sigmoid(gi[:, DIM:2 * DIM] + gh[:, DIM:2 * DIM])
    ng = jnp.tanh(gi[:, 2 * DIM:3 * DIM] + r * gh[:, 2 * DIM:3 * DIM])
    hnew_ref[...] = (1.0 - z) * ng + z * h


def _set2set_body(out_ref, batch_ref, wihst_ref, bihs_ref, whhst_ref, bhhs_ref,
                  w1t_ref, b1_ref, w2t_ref, b2_ref, ge_ref, pred_ref):
    out = out_ref[...]
    mask = batch_ref[...] == lax.broadcasted_iota(jnp.int32, (1, NG), 1)
    maskf = mask.astype(_f32)
    q_star = jnp.zeros((NG, 2 * DIM), dtype=_f32)
    hs = jnp.zeros((NG, DIM), dtype=_f32)
    cs = jnp.zeros((NG, DIM), dtype=_f32)
    for _ in range(3):
        gates = (jnp.dot(q_star, wihst_ref[...], preferred_element_type=_f32) + bihs_ref[...]
                 + jnp.dot(hs, whhst_ref[...], preferred_element_type=_f32) + bhhs_ref[...])
        gi = gates[:, 0:DIM]
        gf = gates[:, DIM:2 * DIM]
        gg = gates[:, 2 * DIM:3 * DIM]
        go = gates[:, 3 * DIM:4 * DIM]
        cs = jax.nn.sigmoid(gf) * cs + jax.nn.sigmoid(gi) * jnp.tanh(gg)
        hs = jax.nn.sigmoid(go) * jnp.tanh(cs)
        q = hs
        qb = jnp.dot(maskf, q, preferred_element_type=_f32, precision=_HI)
        e = jnp.sum(out * qb, axis=1, keepdims=True)
        em = jnp.max(jnp.where(mask, e, -1e30), axis=0, keepdims=True)
        em = jnp.where(em > -1e29, em, 0.0)
        emb = jnp.dot(maskf, em.reshape(NG, 1), preferred_element_type=_f32,
                      precision=_HI)
        a = jnp.exp(e - emb)
        denom = lax.dot_general(maskf, a, (((0,), (0,)), ((), ())),
                                preferred_element_type=_f32, precision=_HI)
        ab = a / (jnp.dot(maskf, denom, preferred_element_type=_f32,
                          precision=_HI) + 1e-16)
        rvec = lax.dot_general(maskf, out * ab, (((0,), (0,)), ((), ())),
                               preferred_element_type=_f32, precision=_HI)
        q_star = jnp.concatenate([q, rvec], axis=1)
    ge = jnp.dot(q_star, w1t_ref[...], preferred_element_type=_f32) + b1_ref[...]
    ge_ref[...] = ge
    pred_ref[...] = jnp.dot(jnp.maximum(ge, 0.0), w2t_ref[...],
                            preferred_element_type=_f32) + b2_ref[...]


def _init_proj(x, w0t, b0r):
    return pl.pallas_call(
        _init_body,
        out_shape=jax.ShapeDtypeStruct((N, DIM), _f32),
    )(x, w0t, b0r)


def _edge_msgs(ea, xj, we1t, be1r, we2t, be2r):
    grid = (N_TILES,)
    return pl.pallas_call(
        _edge_body,
        grid=grid,
        in_specs=[
            pl.BlockSpec((EB, 3), lambda i: (i, 0)),
            pl.BlockSpec((EB, DIM), lambda i: (i, 0)),
            pl.BlockSpec((3, F), lambda i: (0, 0)),
            pl.BlockSpec((1, F), lambda i: (0, 0)),
            pl.BlockSpec((F, DIM * DIM), lambda i: (0, 0)),
            pl.BlockSpec((1, DIM * DIM), lambda i: (0, 0)),
        ],
        out_specs=pl.BlockSpec((EB, DIM), lambda i: (i, 0)),
        out_shape=jax.ShapeDtypeStruct((E, DIM), _f32),
        compiler_params=pltpu.CompilerParams(vmem_limit_bytes=100 * 2**20),
    )(ea, xj, we1t, be1r, we2t, be2r)


def _node_update(aggp, cntp, out, h, wroot, bconvr, wiht, bihr, whht, bhhr):
    return pl.pallas_call(
        _node_body,
        out_shape=jax.ShapeDtypeStruct((N, DIM), _f32),
    )(aggp, cntp, out, h, wroot, bconvr, wiht, bihr, whht, bhhr)


def _set2set(out, batch2, wihst, bihsr, whhst, bhhsr, w1t, b1r, w2t, b2r):
    return pl.pallas_call(
        _set2set_body,
        out_shape=(jax.ShapeDtypeStruct((NG, 2 * DIM), _f32),
                   jax.ShapeDtypeStruct((NG, 1), _f32)),
        compiler_params=pltpu.CompilerParams(vmem_limit_bytes=100 * 2**20),
    )(out, batch2, wihst, bihsr, whhst, bhhsr, w1t, b1r, w2t, b2r)


def _gather_body(table, idx2, out, idx_v, rows_v, semg):
    wid = lax.axis_index("s") * NC + lax.axis_index("c")

    def do_batch(ch0, nch):
        pltpu.sync_copy(idx2.at[pl.ds(ch0, nch)], idx_v.at[pl.ds(0, nch)])
        descs = [pltpu.async_copy(table.at[idx_v.at[b]],
                                  rows_v.at[pl.ds(b * CK, CK)], semg)
                 for b in range(nch)]
        for d in descs:
            d.wait()
        pltpu.sync_copy(rows_v.at[pl.ds(0, nch * CK)],
                        out.at[pl.ds(ch0 * CK, nch * CK)])

    def body(g, _):
        o = g * NW + wid

        @pl.when(o < NOCT)
        def _():
            do_batch(o * OCT, OCT)

        return ()

    lax.fori_loop(0, (NOCT + NW - 1) // NW, body, ())

    @pl.when(wid == 0)
    def _tail():
        do_batch(NOCT * OCT, NCH - NOCT * OCT)


def _sc_gather(table, idx2):
    mesh = plsc.VectorSubcoreMesh(core_axis_name="c", subcore_axis_name="s")
    f = functools.partial(
        pl.kernel,
        out_type=jax.ShapeDtypeStruct((E, DIM), _f32),
        mesh=mesh,
        compiler_params=pltpu.CompilerParams(use_tc_tiling_on_sc=False),
        scratch_types=[
            pltpu.VMEM((OCT, CK), jnp.int32),
            pltpu.VMEM((OCT * CK, DIM), _f32),
            pltpu.SemaphoreType.DMA,
        ],
    )(_gather_body)
    return f(table, idx2)


def _scatter_body(msg, idx2, zeros, out, idx_v, msg_v, agg_sh):
    cid = lax.axis_index("c")
    sid = lax.axis_index("s")
    wid = sid * NC + cid
    pltpu.sync_copy(zeros.at[pl.ds(sid * NPT, NPT)],
                    agg_sh.at[pl.ds(sid * NPT, NPT)])
    plsc.subcore_barrier()

    def do_batch(ch0, nch):
        pltpu.sync_copy(idx2.at[pl.ds(ch0, nch)], idx_v.at[pl.ds(0, nch)])
        pltpu.sync_copy(msg.at[pl.ds(ch0 * CK, nch * CK)],
                        msg_v.at[pl.ds(0, nch * CK)])
        for b in range(nch):
            pltpu.sync_copy(msg_v.at[pl.ds(b * CK, CK)],
                            agg_sh.at[idx_v.at[b]], add=True)

    def body(g, _):
        o = g * NW + wid

        @pl.when(o < NOCT)
        def _():
            do_batch(o * OCT, OCT)

        return ()

    lax.fori_loop(0, (NOCT + NW - 1) // NW, body, ())

    @pl.when(wid == 0)
    def _tail():
        do_batch(NOCT * OCT, NCH - NOCT * OCT)

    plsc.subcore_barrier()
    pltpu.sync_copy(agg_sh.at[pl.ds(sid * NPT, NPT)],
                    out.at[cid, pl.ds(sid * NPT, NPT)])


def _sc_scatter(msg, idx2, zeros):
    mesh = plsc.VectorSubcoreMesh(core_axis_name="c", subcore_axis_name="s")
    f = functools.partial(
        pl.kernel,
        out_type=jax.ShapeDtypeStruct((NC, N_PAD, DIM), _f32),
        mesh=mesh,
        compiler_params=pltpu.CompilerParams(use_tc_tiling_on_sc=False),
        scratch_types=[
            pltpu.VMEM((OCT, CK), jnp.int32),
            pltpu.VMEM((OCT * CK, DIM), _f32),
            pltpu.VMEM_SHARED((N_PAD, DIM), _f32),
        ],
    )(_scatter_body)
    return f(msg, idx2, zeros)


def kernel(x, edge_index, edge_attr, batch, W0, b0, We1, be1, We2, be2,
           Wroot, bconv, Wih, Whh, bih, bhh, Wih_s, Whh_s, bih_s, bhh_s,
           W1, b1, W2, b2):
    src = edge_index[0]
    dst = edge_index[1]
    w0t = W0.T
    we1t = We1.T
    we2t = We2.T
    wiht = Wih.T
    whht = Whh.T
    wihst = Wih_s.T
    whhst = Whh_s.T
    w1t = W1.T
    w2t = W2.T
    b0r = b0.reshape(1, -1)
    be1r = be1.reshape(1, -1)
    be2r = be2.reshape(1, -1)
    bconvr = bconv.reshape(1, -1)
    bihr = bih.reshape(1, -1)
    bhhr = bhh.reshape(1, -1)
    bihsr = bih_s.reshape(1, -1)
    bhhsr = bhh_s.reshape(1, -1)
    b1r = b1.reshape(1, -1)
    b2r = b2.reshape(1, -1)
    batch2 = batch.reshape(N, 1)

    src2 = src.reshape(NCH, CK)
    dst2 = dst.reshape(NCH, CK)
    zero_nd = jnp.zeros((N_PAD, DIM), _f32)
    ones_e = jnp.ones((E, DIM), _f32)

    out = _init_proj(x, w0t, b0r)
    h = out
    cntp = _sc_scatter(ones_e, dst2, zero_nd)
    for _ in range(3):
        xj = _sc_gather(out, src2)
        msg = _edge_msgs(edge_attr, xj, we1t, be1r, we2t, be2r)
        aggp = _sc_scatter(msg, dst2, zero_nd)
        h = _node_update(aggp, cntp, out, h, Wroot, bconvr, wiht, bihr, whht, bhhr)
        out = h
    ge, pred = _set2set(out, batch2, wihst, bihsr, whhst, bhhsr, w1t, b1r, w2t, b2r)
    return pred.reshape(-1), ge
